# baseline (speedup 1.0000x reference)
import jax
import jax.numpy as jnp
from jax.experimental import pallas as pl
from jax.experimental.pallas import tpu as pltpu

N_FILTERS = 16
GRAPHLET = 3
N_GROUPS = 32
REL_DIM = 16
PROJ_DIM = 16
KEY_DIM = 16
BETA = KEY_DIM ** (-0.5)
NQ = N_GROUPS * GRAPHLET
ROWS_PER_STEP = 2


def _fused_kernel(x_ref, q_ref, pose_ref, poso_ref, wk_ref, wp_ref, m_ref,
                  o_ref, pq_ref):
    i = pl.program_id(0)

    @pl.when(i == 0)
    def _init():
        pq_ref[:NQ] = BETA * jax.lax.dot_general(
            q_ref[...], pose_ref[...], (((1,), (1,)), ((), ())),
            preferred_element_type=jnp.float32)
        pq_ref[NQ:] = BETA * jax.lax.dot_general(
            q_ref[...], poso_ref[...], (((1,), (1,)), ((), ())),
            preferred_element_type=jnp.float32)

    d = wk_ref.shape[0]
    qw = BETA * jax.lax.dot_general(
        q_ref[...], wk_ref[...], (((1,), (1,)), ((), ())),
        preferred_element_type=jnp.float32)
    zq = jnp.zeros((NQ, d), dtype=jnp.float32)
    q2 = jnp.concatenate(
        [jnp.concatenate([qw, zq], axis=1),
         jnp.concatenate([zq, qw], axis=1)], axis=0)
    zw = jnp.zeros((d, REL_DIM * PROJ_DIM), dtype=jnp.float32)
    wp_e = jnp.concatenate([wp_ref[...], zw], axis=0)
    wp_o = jnp.concatenate([zw, wp_ref[...]], axis=0)
    rows = jax.lax.broadcasted_iota(jnp.int32, (N_GROUPS, NQ), 0)
    cols = jax.lax.broadcasted_iota(jnp.int32, (N_GROUPS, NQ), 1)
    sels = [(cols == GRAPHLET * rows + g).astype(jnp.float32)
            for g in range(GRAPHLET)]
    for r in range(ROWS_PER_STEP):
        x = x_ref[r]
        logits = jax.lax.dot_general(
            q2, x, (((1,), (1,)), ((), ())),
            preferred_element_type=jnp.float32) + pq_ref[...]
        l1 = logits[:NQ]
        l2 = logits[NQ:]
        mx = jnp.maximum(jnp.max(l1, axis=1, keepdims=True),
                         jnp.max(l2, axis=1, keepdims=True))
        e1 = jnp.exp(l1 - mx)
        e2 = jnp.exp(l2 - mx)
        denom = (jnp.sum(e1, axis=1, keepdims=True)
                 + jnp.sum(e2, axis=1, keepdims=True))
        p1 = jnp.dot(e1, x, preferred_element_type=jnp.float32)
        p2 = jnp.dot(e2, x, preferred_element_type=jnp.float32)
        z = (jnp.dot(p1, wp_e, preferred_element_type=jnp.float32)
             + jnp.dot(p2, wp_o, preferred_element_type=jnp.float32)) / denom
        zs = [jnp.dot(sels[g], z, preferred_element_type=jnp.float32)
              for g in range(GRAPHLET)]
        acc = jnp.zeros((N_GROUPS, N_FILTERS), dtype=jnp.float32)
        for g in range(GRAPHLET):
            for h in range(GRAPHLET):
                w = zs[g] * zs[h]
                acc = acc + jnp.dot(w, m_ref[GRAPHLET * g + h],
                                    preferred_element_type=jnp.float32)
        o_ref[r] = acc


@jax.jit
def kernel(inputs, filters, group_queries, pos_emb, Wk, Wp):
    b, n, d = inputs.shape
    n2 = n // 2
    xp = inputs.reshape(b, n2, 2 * d)
    pos_e = pos_emb[0::2]
    pos_o = pos_emb[1::2]
    m = jnp.repeat(filters.transpose(1, 2, 3, 0), PROJ_DIM, axis=2)
    m = m.reshape(GRAPHLET * GRAPHLET, REL_DIM * PROJ_DIM, N_FILTERS)
    return pl.pallas_call(
        _fused_kernel,
        grid=(b // ROWS_PER_STEP,),
        in_specs=[
            pl.BlockSpec((ROWS_PER_STEP, n2, 2 * d), lambda i: (i, 0, 0)),
            pl.BlockSpec((NQ, KEY_DIM), lambda i: (0, 0)),
            pl.BlockSpec((n2, KEY_DIM), lambda i: (0, 0)),
            pl.BlockSpec((n2, KEY_DIM), lambda i: (0, 0)),
            pl.BlockSpec((d, KEY_DIM), lambda i: (0, 0)),
            pl.BlockSpec((d, REL_DIM * PROJ_DIM), lambda i: (0, 0)),
            pl.BlockSpec((GRAPHLET * GRAPHLET, REL_DIM * PROJ_DIM, N_FILTERS),
                         lambda i: (0, 0, 0)),
        ],
        out_specs=pl.BlockSpec((ROWS_PER_STEP, N_GROUPS, N_FILTERS),
                               lambda i: (i, 0, 0)),
        out_shape=jax.ShapeDtypeStruct((b, N_GROUPS, N_FILTERS), jnp.float32),
        scratch_shapes=[pltpu.VMEM((2 * NQ, n2), jnp.float32)],
    )(xp, group_queries, pos_e, pos_o, Wk, Wp, m)


# speedup vs baseline: 1.3723x; 1.3723x over previous
import jax
import jax.numpy as jnp
from jax.experimental import pallas as pl
from jax.experimental.pallas import tpu as pltpu

N_FILTERS = 16
GRAPHLET = 3
N_GROUPS = 32
REL_DIM = 16
PROJ_DIM = 16
KEY_DIM = 16
BETA = KEY_DIM ** (-0.5)
NQ = N_GROUPS * GRAPHLET


def _fused_kernel(x_hbm, q_ref, pos_ref, wk_ref, wp_ref, m_ref, o_ref,
                  xbuf, pq_ref, sem):
    i = pl.program_id(0)
    nb = pl.num_programs(0)

    @pl.when(i == 0)
    def _init():
        pq_ref[...] = BETA * jax.lax.dot_general(
            q_ref[...], pos_ref[...], (((1,), (1,)), ((), ())),
            preferred_element_type=jnp.float32)
        pltpu.make_async_copy(x_hbm.at[0], xbuf.at[0], sem.at[0]).start()

    @pl.when(i + 1 < nb)
    def _prefetch():
        pltpu.make_async_copy(
            x_hbm.at[i + 1], xbuf.at[(i + 1) % 2], sem.at[(i + 1) % 2]).start()

    pltpu.make_async_copy(x_hbm.at[i], xbuf.at[i % 2], sem.at[i % 2]).wait()
    x = xbuf[i % 2]
    qw = BETA * jax.lax.dot_general(
        q_ref[...], wk_ref[...], (((1,), (1,)), ((), ())),
        preferred_element_type=jnp.float32)
    logits = jax.lax.dot_general(
        qw, x, (((1,), (1,)), ((), ())),
        preferred_element_type=jnp.float32) + pq_ref[...]
    mx = jnp.max(logits, axis=1, keepdims=True)
    e = jnp.exp(logits - mx)
    denom = jnp.sum(e, axis=1, keepdims=True)
    attn = jnp.dot(e, x, preferred_element_type=jnp.float32) / denom
    rows = jax.lax.broadcasted_iota(jnp.int32, (N_GROUPS, NQ), 0)
    cols = jax.lax.broadcasted_iota(jnp.int32, (N_GROUPS, NQ), 1)
    zs = []
    for g in range(GRAPHLET):
        sel = (cols == GRAPHLET * rows + g).astype(jnp.float32)
        attn_g = jnp.dot(sel, attn, preferred_element_type=jnp.float32)
        zs.append(jnp.dot(attn_g, wp_ref[...],
                          preferred_element_type=jnp.float32))
    acc = jnp.zeros((N_GROUPS, N_FILTERS), dtype=jnp.float32)
    for g in range(GRAPHLET):
        for h in range(GRAPHLET):
            w = zs[g] * zs[h]
            acc = acc + jnp.dot(w, m_ref[GRAPHLET * g + h],
                                preferred_element_type=jnp.float32)
    o_ref[0] = acc


@jax.jit
def kernel(inputs, filters, group_queries, pos_emb, Wk, Wp):
    b, n, d = inputs.shape
    m = jnp.repeat(filters.transpose(1, 2, 3, 0), PROJ_DIM, axis=2)
    m = m.reshape(GRAPHLET * GRAPHLET, REL_DIM * PROJ_DIM, N_FILTERS)
    return pl.pallas_call(
        _fused_kernel,
        grid=(b,),
        in_specs=[
            pl.BlockSpec(memory_space=pl.ANY),
            pl.BlockSpec((NQ, KEY_DIM), lambda i: (0, 0)),
            pl.BlockSpec((n, KEY_DIM), lambda i: (0, 0)),
            pl.BlockSpec((d, KEY_DIM), lambda i: (0, 0)),
            pl.BlockSpec((d, REL_DIM * PROJ_DIM), lambda i: (0, 0)),
            pl.BlockSpec((GRAPHLET * GRAPHLET, REL_DIM * PROJ_DIM, N_FILTERS),
                         lambda i: (0, 0, 0)),
        ],
        out_specs=pl.BlockSpec((1, N_GROUPS, N_FILTERS), lambda i: (i, 0, 0)),
        out_shape=jax.ShapeDtypeStruct((b, N_GROUPS, N_FILTERS), jnp.float32),
        scratch_shapes=[
            pltpu.VMEM((2, n, d), jnp.float32),
            pltpu.VMEM((NQ, n), jnp.float32),
            pltpu.SemaphoreType.DMA((2,)),
        ],
    )(inputs, group_queries, pos_emb, Wk, Wp, m)


# speedup vs baseline: 1.4091x; 1.0269x over previous
import jax
import jax.numpy as jnp
from jax.experimental import pallas as pl
from jax.experimental.pallas import tpu as pltpu

N_FILTERS = 16
GRAPHLET = 3
N_GROUPS = 32
REL_DIM = 16
PROJ_DIM = 16
KEY_DIM = 16
BETA = KEY_DIM ** (-0.5)
NQ = N_GROUPS * GRAPHLET
ROWS_PER_STEP = 2


def _fused_kernel(x_ref, q_ref, pos_ref, wk_ref, wp_ref, m_ref, o_ref, pq_ref):
    i = pl.program_id(0)

    @pl.when(i == 0)
    def _init():
        pq_ref[...] = BETA * jax.lax.dot_general(
            q_ref[...], pos_ref[...], (((1,), (1,)), ((), ())),
            preferred_element_type=jnp.float32)

    qw = (BETA * jax.lax.dot_general(
        q_ref[...], wk_ref[...], (((1,), (1,)), ((), ())),
        preferred_element_type=jnp.float32)).astype(jnp.bfloat16)
    rows = jax.lax.broadcasted_iota(jnp.int32, (N_GROUPS, NQ), 0)
    cols = jax.lax.broadcasted_iota(jnp.int32, (N_GROUPS, NQ), 1)
    sels = [(cols == GRAPHLET * rows + g).astype(jnp.float32)
            for g in range(GRAPHLET)]
    for r in range(ROWS_PER_STEP):
        x = x_ref[r]
        logits = jax.lax.dot_general(
            qw, x, (((1,), (1,)), ((), ())),
            preferred_element_type=jnp.float32) + pq_ref[...]
        mx = jnp.max(logits, axis=1, keepdims=True)
        e = jnp.exp(logits - mx)
        denom = jnp.sum(e, axis=1, keepdims=True)
        attn = jnp.dot(e.astype(jnp.bfloat16), x,
                       preferred_element_type=jnp.float32) / denom
        zs = []
        for g in range(GRAPHLET):
            attn_g = jnp.dot(sels[g], attn, preferred_element_type=jnp.float32)
            zs.append(jnp.dot(attn_g, wp_ref[...],
                              preferred_element_type=jnp.float32))
        acc = jnp.zeros((N_GROUPS, N_FILTERS), dtype=jnp.float32)
        for g in range(GRAPHLET):
            for h in range(GRAPHLET):
                w = zs[g] * zs[h]
                acc = acc + jnp.dot(w, m_ref[GRAPHLET * g + h],
                                    preferred_element_type=jnp.float32)
        o_ref[r] = acc


@jax.jit
def kernel(inputs, filters, group_queries, pos_emb, Wk, Wp):
    b, n, d = inputs.shape
    xb = inputs.astype(jnp.bfloat16)
    qb = group_queries.astype(jnp.bfloat16)
    posb = pos_emb.astype(jnp.bfloat16)
    wkb = Wk.astype(jnp.bfloat16)
    m = jnp.repeat(filters.transpose(1, 2, 3, 0), PROJ_DIM, axis=2)
    m = m.reshape(GRAPHLET * GRAPHLET, REL_DIM * PROJ_DIM, N_FILTERS)
    return pl.pallas_call(
        _fused_kernel,
        grid=(b // ROWS_PER_STEP,),
        in_specs=[
            pl.BlockSpec((ROWS_PER_STEP, n, d), lambda i: (i, 0, 0)),
            pl.BlockSpec((NQ, KEY_DIM), lambda i: (0, 0)),
            pl.BlockSpec((n, KEY_DIM), lambda i: (0, 0)),
            pl.BlockSpec((d, KEY_DIM), lambda i: (0, 0)),
            pl.BlockSpec((d, REL_DIM * PROJ_DIM), lambda i: (0, 0)),
            pl.BlockSpec((GRAPHLET * GRAPHLET, REL_DIM * PROJ_DIM, N_FILTERS),
                         lambda i: (0, 0, 0)),
        ],
        out_specs=pl.BlockSpec((ROWS_PER_STEP, N_GROUPS, N_FILTERS),
                               lambda i: (i, 0, 0)),
        out_shape=jax.ShapeDtypeStruct((b, N_GROUPS, N_FILTERS), jnp.float32),
        scratch_shapes=[pltpu.VMEM((NQ, n), jnp.float32)],
    )(xb, qb, posb, wkb, Wp, m)
